# Initial kernel scaffold; baseline (speedup 1.0000x reference)
import jax
import jax.numpy as jnp
from jax.experimental import pallas as pl


def kernel(roi_features, union_features, rel_pair_idxs, W_obj_down, W_rel_down, W_n2e, W_e2n, W_obj_cls, b_obj_cls, W_rel_cls, b_rel_cls):
    raise NotImplementedError("write your pallas kernel here")


# speedup vs baseline: 3.7549x; 3.7549x over previous
import functools

import jax
import jax.numpy as jnp
from jax import lax
from jax.experimental import pallas as pl
from jax.experimental.pallas import tpu as pltpu
from jax.experimental.pallas import tpu_sc as plsc

N_OBJ = 10000
N_REL = 160000
HID = 128

NC = 2
NS = 16
NW = NC * NS
EPW = N_REL // NW
CH = 125
NCH = EPW // CH
RPT = N_OBJ // NS
DEGW = 16

_sc_mesh = plsc.VectorSubcoreMesh(core_axis_name="c", subcore_axis_name="s")
_SC_PARAMS = pltpu.CompilerParams(use_tc_tiling_on_sc=False)


@functools.partial(
    pl.kernel,
    out_type=(
        jax.ShapeDtypeStruct((N_REL, HID), jnp.float32),
        jax.ShapeDtypeStruct((N_REL, HID), jnp.float32),
    ),
    mesh=_sc_mesh,
    scratch_types=[
        pltpu.VMEM((NCH, CH), jnp.int32),
        pltpu.VMEM((NCH, CH), jnp.int32),
        pltpu.VMEM((CH, HID), jnp.float32),
        pltpu.SemaphoreType.DMA,
    ],
    compiler_params=_SC_PARAMS,
)
def _sc_gather(p_hbm, src_hbm, dst_hbm, gs_hbm, gd_hbm, idxs_v, idxd_v,
               rows_v, sem):
    wid = lax.axis_index("s") * NC + lax.axis_index("c")
    pltpu.sync_copy(src_hbm.at[pl.ds(wid * NCH, NCH)], idxs_v)
    pltpu.sync_copy(dst_hbm.at[pl.ds(wid * NCH, NCH)], idxd_v)

    def chunk(j, carry):
        off = wid * EPW + j * CH
        pltpu.async_copy(p_hbm.at[idxs_v.at[j]], rows_v, sem).wait()
        pltpu.sync_copy(rows_v, gs_hbm.at[pl.ds(off, CH)])
        pltpu.async_copy(p_hbm.at[idxd_v.at[j]], rows_v, sem).wait()
        pltpu.sync_copy(rows_v, gd_hbm.at[pl.ds(off, CH)])
        return carry

    lax.fori_loop(0, NCH, chunk, 0)


@functools.partial(
    pl.kernel,
    out_type=(
        jax.ShapeDtypeStruct((NC, N_OBJ, HID), jnp.float32),
        jax.ShapeDtypeStruct((NC, N_OBJ, DEGW), jnp.float32),
    ),
    mesh=_sc_mesh,
    scratch_types=[
        pltpu.VMEM((NCH, CH), jnp.int32),
        pltpu.VMEM((NCH, CH), jnp.int32),
        pltpu.VMEM((CH, HID), jnp.float32),
        pltpu.VMEM((CH, DEGW), jnp.float32),
        pltpu.VMEM_SHARED((N_OBJ, HID), jnp.float32),
        pltpu.VMEM_SHARED((N_OBJ, DEGW), jnp.float32),
    ],
    compiler_params=_SC_PARAMS,
)
def _sc_scatter(msg_hbm, src_hbm, dst_hbm, z128_hbm, z16_hbm, ones16_hbm,
                pout_hbm, dout_hbm, idxs_v, idxd_v, rows_v, s16_v, acc_sh,
                deg_sh):
    cid = lax.axis_index("c")
    sid = lax.axis_index("s")
    wid = sid * NC + cid

    pltpu.sync_copy(z128_hbm, rows_v)
    pltpu.sync_copy(z16_hbm, s16_v)
    for q in range(RPT // CH):
        r0 = sid * RPT + q * CH
        pltpu.sync_copy(rows_v, acc_sh.at[pl.ds(r0, CH)])
        pltpu.sync_copy(s16_v, deg_sh.at[pl.ds(r0, CH)])
    plsc.subcore_barrier()

    pltpu.sync_copy(src_hbm.at[pl.ds(wid * NCH, NCH)], idxs_v)
    pltpu.sync_copy(dst_hbm.at[pl.ds(wid * NCH, NCH)], idxd_v)
    pltpu.sync_copy(ones16_hbm, s16_v)

    def chunk(j, carry):
        pltpu.sync_copy(msg_hbm.at[pl.ds(wid * EPW + j * CH, CH)], rows_v)
        pltpu.sync_copy(rows_v, acc_sh.at[idxs_v.at[j]], add=True)
        pltpu.sync_copy(rows_v, acc_sh.at[idxd_v.at[j]], add=True)
        pltpu.sync_copy(s16_v, deg_sh.at[idxs_v.at[j]], add=True)
        pltpu.sync_copy(s16_v, deg_sh.at[idxd_v.at[j]], add=True)
        return carry

    lax.fori_loop(0, NCH, chunk, 0)
    plsc.subcore_barrier()

    for q in range(RPT // CH):
        r0 = sid * RPT + q * CH
        pltpu.sync_copy(acc_sh.at[pl.ds(r0, CH)], rows_v)
        pltpu.sync_copy(rows_v, pout_hbm.at[cid].at[pl.ds(r0, CH)])
        pltpu.sync_copy(deg_sh.at[pl.ds(r0, CH)], s16_v)
        pltpu.sync_copy(s16_v, dout_hbm.at[cid].at[pl.ds(r0, CH)])


_OBJ_BLK = 1000
_REL_BLK = 2000


def _mm(a, b):
    return jax.lax.dot_general(a, b, (((1,), (0,)), ((), ())),
                               preferred_element_type=jnp.float32)


def _prologue_body(roi_ref, wod_ref, wn2e_ref, objh_ref, p_ref):
    h = jnp.maximum(_mm(roi_ref[...], wod_ref[...]), 0.0)
    objh_ref[...] = h
    p_ref[...] = _mm(h, wn2e_ref[...])


def _edge0_body(u_ref, wrd_ref, gs_ref, gd_ref, we2n_ref, relh_ref, msg_ref):
    rh = jnp.maximum(_mm(u_ref[...], wrd_ref[...]), 0.0)
    t = rh + jnp.maximum(gs_ref[...] + gd_ref[...], 0.0)
    relh_ref[...] = t
    msg_ref[...] = jnp.maximum(_mm(t, we2n_ref[...]), 0.0)


def _edge1_body(relh_ref, gs_ref, gd_ref, we2n_ref, wcls_ref, b_ref,
                msg_ref, logit_ref):
    t = relh_ref[...] + jnp.maximum(gs_ref[...] + gd_ref[...], 0.0)
    msg_ref[...] = jnp.maximum(_mm(t, we2n_ref[...]), 0.0)
    logit_ref[...] = _mm(t, wcls_ref[...]) + b_ref[...]


def _node_update(obj_ref, p_ref, dg_ref):
    agg = p_ref[0] + p_ref[1]
    deg = jnp.maximum(dg_ref[0][:, 0:1] + dg_ref[1][:, 0:1], 1.0)
    return obj_ref[...] + agg / deg


def _node0_body(obj_ref, p_ref, dg_ref, wn2e_ref, objo_ref, po_ref):
    o = _node_update(obj_ref, p_ref, dg_ref)
    objo_ref[...] = o
    po_ref[...] = _mm(o, wn2e_ref[...])


def _node1_body(obj_ref, p_ref, dg_ref, wcls_ref, b_ref, logit_ref):
    o = _node_update(obj_ref, p_ref, dg_ref)
    logit_ref[...] = _mm(o, wcls_ref[...]) + b_ref[...]


def _row_spec(blk, width):
    return pl.BlockSpec((blk, width), lambda i: (i, 0))


def _full_spec(shape):
    ndim = len(shape)
    return pl.BlockSpec(shape, lambda i: (0,) * ndim)


def _part_spec(blk, width):
    return pl.BlockSpec((NC, blk, width), lambda i: (0, i, 0))


_TC_PARAMS = pltpu.CompilerParams(dimension_semantics=("parallel",))


def _tc_prologue(roi, wod, wn2e):
    return pl.pallas_call(
        _prologue_body,
        grid=(N_OBJ // _OBJ_BLK,),
        in_specs=[_row_spec(_OBJ_BLK, HID), _full_spec(wod.shape),
                  _full_spec(wn2e.shape)],
        out_specs=(_row_spec(_OBJ_BLK, HID), _row_spec(_OBJ_BLK, HID)),
        out_shape=(jax.ShapeDtypeStruct((N_OBJ, HID), jnp.float32),
                   jax.ShapeDtypeStruct((N_OBJ, HID), jnp.float32)),
        compiler_params=_TC_PARAMS,
    )(roi, wod, wn2e)


def _tc_edge0(union, wrd, gs, gd, we2n):
    return pl.pallas_call(
        _edge0_body,
        grid=(N_REL // _REL_BLK,),
        in_specs=[_row_spec(_REL_BLK, HID), _full_spec(wrd.shape),
                  _row_spec(_REL_BLK, HID), _row_spec(_REL_BLK, HID),
                  _full_spec(we2n.shape)],
        out_specs=(_row_spec(_REL_BLK, HID), _row_spec(_REL_BLK, HID)),
        out_shape=(jax.ShapeDtypeStruct((N_REL, HID), jnp.float32),
                   jax.ShapeDtypeStruct((N_REL, HID), jnp.float32)),
        compiler_params=_TC_PARAMS,
    )(union, wrd, gs, gd, we2n)


def _tc_edge1(relh, gs, gd, we2n, wcls, b):
    ncls = wcls.shape[1]
    return pl.pallas_call(
        _edge1_body,
        grid=(N_REL // _REL_BLK,),
        in_specs=[_row_spec(_REL_BLK, HID), _row_spec(_REL_BLK, HID),
                  _row_spec(_REL_BLK, HID), _full_spec(we2n.shape),
                  _full_spec(wcls.shape), _full_spec(b.shape)],
        out_specs=(_row_spec(_REL_BLK, HID), _row_spec(_REL_BLK, ncls)),
        out_shape=(jax.ShapeDtypeStruct((N_REL, HID), jnp.float32),
                   jax.ShapeDtypeStruct((N_REL, ncls), jnp.float32)),
        compiler_params=_TC_PARAMS,
    )(relh, gs, gd, we2n, wcls, b)


def _tc_node0(obj_h, parts, degp, wn2e):
    return pl.pallas_call(
        _node0_body,
        grid=(N_OBJ // _OBJ_BLK,),
        in_specs=[_row_spec(_OBJ_BLK, HID), _part_spec(_OBJ_BLK, HID),
                  _part_spec(_OBJ_BLK, DEGW), _full_spec(wn2e.shape)],
        out_specs=(_row_spec(_OBJ_BLK, HID), _row_spec(_OBJ_BLK, HID)),
        out_shape=(jax.ShapeDtypeStruct((N_OBJ, HID), jnp.float32),
                   jax.ShapeDtypeStruct((N_OBJ, HID), jnp.float32)),
        compiler_params=_TC_PARAMS,
    )(obj_h, parts, degp, wn2e)


def _tc_node1(obj_h, parts, degp, wcls, b):
    ncls = wcls.shape[1]
    return pl.pallas_call(
        _node1_body,
        grid=(N_OBJ // _OBJ_BLK,),
        in_specs=[_row_spec(_OBJ_BLK, HID), _part_spec(_OBJ_BLK, HID),
                  _part_spec(_OBJ_BLK, DEGW), _full_spec(wcls.shape),
                  _full_spec(b.shape)],
        out_specs=_row_spec(_OBJ_BLK, ncls),
        out_shape=jax.ShapeDtypeStruct((N_OBJ, ncls), jnp.float32),
        compiler_params=_TC_PARAMS,
    )(obj_h, parts, degp, wcls, b)


def kernel(roi_features, union_features, rel_pair_idxs, W_obj_down,
           W_rel_down, W_n2e, W_e2n, W_obj_cls, b_obj_cls, W_rel_cls,
           b_rel_cls):
    src = rel_pair_idxs[:, 0].astype(jnp.int32).reshape(NW * NCH, CH)
    dst = rel_pair_idxs[:, 1].astype(jnp.int32).reshape(NW * NCH, CH)
    z128 = jnp.zeros((CH, HID), jnp.float32)
    z16 = jnp.zeros((CH, DEGW), jnp.float32)
    ones16 = jnp.ones((CH, DEGW), jnp.float32)
    b_obj = b_obj_cls.reshape(1, -1)
    b_rel = b_rel_cls.reshape(1, -1)

    obj_h, p = _tc_prologue(roi_features, W_obj_down, W_n2e)

    gs, gd = _sc_gather(p, src, dst)
    rel_h, msg = _tc_edge0(union_features, W_rel_down, gs, gd, W_e2n)
    parts, degp = _sc_scatter(msg, src, dst, z128, z16, ones16)
    obj_h, p = _tc_node0(obj_h, parts, degp, W_n2e)

    gs, gd = _sc_gather(p, src, dst)
    msg, rel_logits = _tc_edge1(rel_h, gs, gd, W_e2n, W_rel_cls, b_rel)
    parts, degp = _sc_scatter(msg, src, dst, z128, z16, ones16)
    obj_logits = _tc_node1(obj_h, parts, degp, W_obj_cls, b_obj)

    return (obj_logits, rel_logits)


# speedup vs baseline: 4.6327x; 1.2338x over previous
import functools

import jax
import jax.numpy as jnp
from jax import lax
from jax.experimental import pallas as pl
from jax.experimental.pallas import tpu as pltpu
from jax.experimental.pallas import tpu_sc as plsc

N_OBJ = 10000
N_REL = 160000
HID = 128

NC = 2
NS = 16
NW = NC * NS
EPW = N_REL // NW
CH = 125
NCH = EPW // CH
RPT = N_OBJ // NS
DEGW = 16

_sc_mesh = plsc.VectorSubcoreMesh(core_axis_name="c", subcore_axis_name="s")
_SC_PARAMS = pltpu.CompilerParams(use_tc_tiling_on_sc=False)


@functools.partial(
    pl.kernel,
    out_type=(
        jax.ShapeDtypeStruct((N_REL, HID), jnp.float32),
        jax.ShapeDtypeStruct((N_REL, HID), jnp.float32),
    ),
    mesh=_sc_mesh,
    scratch_types=[
        pltpu.VMEM((NCH, CH), jnp.int32),
        pltpu.VMEM((NCH, CH), jnp.int32),
        pltpu.VMEM((CH, HID), jnp.float32),
        pltpu.VMEM((CH, HID), jnp.float32),
        pltpu.VMEM((CH, HID), jnp.float32),
        pltpu.VMEM((CH, HID), jnp.float32),
        pltpu.SemaphoreType.DMA,
        pltpu.SemaphoreType.DMA,
        pltpu.SemaphoreType.DMA,
        pltpu.SemaphoreType.DMA,
        pltpu.SemaphoreType.DMA,
        pltpu.SemaphoreType.DMA,
        pltpu.SemaphoreType.DMA,
        pltpu.SemaphoreType.DMA,
    ],
    compiler_params=_SC_PARAMS,
)
def _sc_gather(p_hbm, src_hbm, dst_hbm, gs_hbm, gd_hbm, idxs_v, idxd_v,
               bufa0, bufb0, bufa1, bufb1, gsa0, gsb0, gsa1, gsb1,
               wsa0, wsb0, wsa1, wsb1):
    wid = lax.axis_index("s") * NC + lax.axis_index("c")
    pltpu.sync_copy(src_hbm.at[pl.ds(wid * NCH, NCH)], idxs_v)
    pltpu.sync_copy(dst_hbm.at[pl.ds(wid * NCH, NCH)], idxd_v)

    def step(j, bufa, bufb, gsa, gsb, wsa, wsb):
        off = wid * EPW + j * CH

        @pl.when(j >= 2)
        def _():
            pltpu.make_async_copy(bufa, gs_hbm.at[pl.ds(off, CH)],
                                  wsa).wait()
            pltpu.make_async_copy(bufb, gd_hbm.at[pl.ds(off, CH)],
                                  wsb).wait()

        ga = pltpu.async_copy(p_hbm.at[idxs_v.at[j]], bufa, gsa)
        gb = pltpu.async_copy(p_hbm.at[idxd_v.at[j]], bufb, gsb)
        ga.wait()
        pltpu.async_copy(bufa, gs_hbm.at[pl.ds(off, CH)], wsa)
        gb.wait()
        pltpu.async_copy(bufb, gd_hbm.at[pl.ds(off, CH)], wsb)

    def chunk2(jj, carry):
        step(2 * jj, bufa0, bufb0, gsa0, gsb0, wsa0, wsb0)
        step(2 * jj + 1, bufa1, bufb1, gsa1, gsb1, wsa1, wsb1)
        return carry

    lax.fori_loop(0, NCH // 2, chunk2, 0)
    last = wid * EPW + (NCH - 1) * CH
    pltpu.make_async_copy(bufa0, gs_hbm.at[pl.ds(last, CH)], wsa0).wait()
    pltpu.make_async_copy(bufb0, gd_hbm.at[pl.ds(last, CH)], wsb0).wait()
    pltpu.make_async_copy(bufa1, gs_hbm.at[pl.ds(last, CH)], wsa1).wait()
    pltpu.make_async_copy(bufb1, gd_hbm.at[pl.ds(last, CH)], wsb1).wait()


@functools.partial(
    pl.kernel,
    out_type=(
        jax.ShapeDtypeStruct((NC, N_OBJ, HID), jnp.float32),
        jax.ShapeDtypeStruct((NC, N_OBJ, DEGW), jnp.float32),
    ),
    mesh=_sc_mesh,
    scratch_types=[
        pltpu.VMEM((2, CH), jnp.int32),
        pltpu.VMEM((2, CH), jnp.int32),
        pltpu.VMEM((CH, HID), jnp.float32),
        pltpu.VMEM((CH, HID), jnp.float32),
        pltpu.VMEM((CH, DEGW), jnp.float32),
        pltpu.VMEM_SHARED((N_OBJ, HID), jnp.float32),
        pltpu.VMEM_SHARED((N_OBJ, DEGW), jnp.float32),
        pltpu.SemaphoreType.DMA,
        pltpu.SemaphoreType.DMA,
    ],
    compiler_params=_SC_PARAMS,
)
def _sc_scatter(msg_hbm, src_hbm, dst_hbm, z128_hbm, z16_hbm, ones16_hbm,
                pout_hbm, dout_hbm, idx0_v, idx1_v, row0_v, row1_v, s16_v,
                acc_sh, deg_sh, lsem0, lsem1):
    cid = lax.axis_index("c")
    sid = lax.axis_index("s")
    wid = sid * NC + cid

    pltpu.sync_copy(z128_hbm, row0_v)
    pltpu.sync_copy(z16_hbm, s16_v)
    for q in range(RPT // CH):
        r0 = sid * RPT + q * CH
        pltpu.sync_copy(row0_v, acc_sh.at[pl.ds(r0, CH)])
        pltpu.sync_copy(s16_v, deg_sh.at[pl.ds(r0, CH)])
    plsc.subcore_barrier()

    pltpu.sync_copy(ones16_hbm, s16_v)

    def load(j, rbuf, ibuf, sem):
        pltpu.async_copy(msg_hbm.at[pl.ds(wid * EPW + j * CH, CH)], rbuf,
                         sem)
        pltpu.async_copy(src_hbm.at[pl.ds(wid * NCH + j, 1)], ibuf.at[0:1],
                         sem)
        pltpu.async_copy(dst_hbm.at[pl.ds(wid * NCH + j, 1)], ibuf.at[1:2],
                         sem)

    def wait(rbuf, ibuf, sem):
        pltpu.make_async_copy(msg_hbm.at[pl.ds(0, CH)], rbuf, sem).wait()
        pltpu.make_async_copy(src_hbm.at[pl.ds(0, 1)], ibuf.at[0:1],
                              sem).wait()
        pltpu.make_async_copy(src_hbm.at[pl.ds(0, 1)], ibuf.at[1:2],
                              sem).wait()

    def scat(rbuf, ibuf):
        pltpu.sync_copy(rbuf, acc_sh.at[ibuf.at[0]], add=True)
        pltpu.sync_copy(rbuf, acc_sh.at[ibuf.at[1]], add=True)
        pltpu.sync_copy(s16_v, deg_sh.at[ibuf.at[0]], add=True)
        pltpu.sync_copy(s16_v, deg_sh.at[ibuf.at[1]], add=True)

    load(0, row0_v, idx0_v, lsem0)

    def chunk2(jj, carry):
        j = 2 * jj
        load(j + 1, row1_v, idx1_v, lsem1)
        wait(row0_v, idx0_v, lsem0)
        scat(row0_v, idx0_v)

        @pl.when(j + 2 < NCH)
        def _():
            load(j + 2, row0_v, idx0_v, lsem0)

        wait(row1_v, idx1_v, lsem1)
        scat(row1_v, idx1_v)
        return carry

    lax.fori_loop(0, NCH // 2, chunk2, 0)
    plsc.subcore_barrier()

    for q in range(RPT // CH):
        r0 = sid * RPT + q * CH
        pltpu.sync_copy(acc_sh.at[pl.ds(r0, CH)], row0_v)
        pltpu.sync_copy(row0_v, pout_hbm.at[cid].at[pl.ds(r0, CH)])
        pltpu.sync_copy(deg_sh.at[pl.ds(r0, CH)], s16_v)
        pltpu.sync_copy(s16_v, dout_hbm.at[cid].at[pl.ds(r0, CH)])


_OBJ_BLK = 1000
_REL_BLK = 2000


def _mm(a, b):
    return jax.lax.dot_general(a, b, (((1,), (0,)), ((), ())),
                               preferred_element_type=jnp.float32)


def _prologue_body(roi_ref, wod_ref, wn2e_ref, objh_ref, p_ref):
    h = jnp.maximum(_mm(roi_ref[...], wod_ref[...]), 0.0)
    objh_ref[...] = h
    p_ref[...] = _mm(h, wn2e_ref[...])


def _edge0_body(u_ref, wrd_ref, gs_ref, gd_ref, we2n_ref, relh_ref, msg_ref):
    rh = jnp.maximum(_mm(u_ref[...], wrd_ref[...]), 0.0)
    t = rh + jnp.maximum(gs_ref[...] + gd_ref[...], 0.0)
    relh_ref[...] = t
    msg_ref[...] = jnp.maximum(_mm(t, we2n_ref[...]), 0.0)


def _edge1_body(relh_ref, gs_ref, gd_ref, we2n_ref, wcls_ref, b_ref,
                msg_ref, logit_ref):
    t = relh_ref[...] + jnp.maximum(gs_ref[...] + gd_ref[...], 0.0)
    msg_ref[...] = jnp.maximum(_mm(t, we2n_ref[...]), 0.0)
    logit_ref[...] = _mm(t, wcls_ref[...]) + b_ref[...]


def _node_update(obj_ref, p_ref, dg_ref):
    agg = p_ref[0] + p_ref[1]
    deg = jnp.maximum(dg_ref[0][:, 0:1] + dg_ref[1][:, 0:1], 1.0)
    return obj_ref[...] + agg / deg


def _node0_body(obj_ref, p_ref, dg_ref, wn2e_ref, objo_ref, po_ref):
    o = _node_update(obj_ref, p_ref, dg_ref)
    objo_ref[...] = o
    po_ref[...] = _mm(o, wn2e_ref[...])


def _node1_body(obj_ref, p_ref, dg_ref, wcls_ref, b_ref, logit_ref):
    o = _node_update(obj_ref, p_ref, dg_ref)
    logit_ref[...] = _mm(o, wcls_ref[...]) + b_ref[...]


def _row_spec(blk, width):
    return pl.BlockSpec((blk, width), lambda i: (i, 0))


def _full_spec(shape):
    ndim = len(shape)
    return pl.BlockSpec(shape, lambda i: (0,) * ndim)


def _part_spec(blk, width):
    return pl.BlockSpec((NC, blk, width), lambda i: (0, i, 0))


_TC_PARAMS = pltpu.CompilerParams(dimension_semantics=("parallel",))


def _tc_prologue(roi, wod, wn2e):
    return pl.pallas_call(
        _prologue_body,
        grid=(N_OBJ // _OBJ_BLK,),
        in_specs=[_row_spec(_OBJ_BLK, HID), _full_spec(wod.shape),
                  _full_spec(wn2e.shape)],
        out_specs=(_row_spec(_OBJ_BLK, HID), _row_spec(_OBJ_BLK, HID)),
        out_shape=(jax.ShapeDtypeStruct((N_OBJ, HID), jnp.float32),
                   jax.ShapeDtypeStruct((N_OBJ, HID), jnp.float32)),
        compiler_params=_TC_PARAMS,
    )(roi, wod, wn2e)


def _tc_edge0(union, wrd, gs, gd, we2n):
    return pl.pallas_call(
        _edge0_body,
        grid=(N_REL // _REL_BLK,),
        in_specs=[_row_spec(_REL_BLK, HID), _full_spec(wrd.shape),
                  _row_spec(_REL_BLK, HID), _row_spec(_REL_BLK, HID),
                  _full_spec(we2n.shape)],
        out_specs=(_row_spec(_REL_BLK, HID), _row_spec(_REL_BLK, HID)),
        out_shape=(jax.ShapeDtypeStruct((N_REL, HID), jnp.float32),
                   jax.ShapeDtypeStruct((N_REL, HID), jnp.float32)),
        compiler_params=_TC_PARAMS,
    )(union, wrd, gs, gd, we2n)


def _tc_edge1(relh, gs, gd, we2n, wcls, b):
    ncls = wcls.shape[1]
    return pl.pallas_call(
        _edge1_body,
        grid=(N_REL // _REL_BLK,),
        in_specs=[_row_spec(_REL_BLK, HID), _row_spec(_REL_BLK, HID),
                  _row_spec(_REL_BLK, HID), _full_spec(we2n.shape),
                  _full_spec(wcls.shape), _full_spec(b.shape)],
        out_specs=(_row_spec(_REL_BLK, HID), _row_spec(_REL_BLK, ncls)),
        out_shape=(jax.ShapeDtypeStruct((N_REL, HID), jnp.float32),
                   jax.ShapeDtypeStruct((N_REL, ncls), jnp.float32)),
        compiler_params=_TC_PARAMS,
    )(relh, gs, gd, we2n, wcls, b)


def _tc_node0(obj_h, parts, degp, wn2e):
    return pl.pallas_call(
        _node0_body,
        grid=(N_OBJ // _OBJ_BLK,),
        in_specs=[_row_spec(_OBJ_BLK, HID), _part_spec(_OBJ_BLK, HID),
                  _part_spec(_OBJ_BLK, DEGW), _full_spec(wn2e.shape)],
        out_specs=(_row_spec(_OBJ_BLK, HID), _row_spec(_OBJ_BLK, HID)),
        out_shape=(jax.ShapeDtypeStruct((N_OBJ, HID), jnp.float32),
                   jax.ShapeDtypeStruct((N_OBJ, HID), jnp.float32)),
        compiler_params=_TC_PARAMS,
    )(obj_h, parts, degp, wn2e)


def _tc_node1(obj_h, parts, degp, wcls, b):
    ncls = wcls.shape[1]
    return pl.pallas_call(
        _node1_body,
        grid=(N_OBJ // _OBJ_BLK,),
        in_specs=[_row_spec(_OBJ_BLK, HID), _part_spec(_OBJ_BLK, HID),
                  _part_spec(_OBJ_BLK, DEGW), _full_spec(wcls.shape),
                  _full_spec(b.shape)],
        out_specs=_row_spec(_OBJ_BLK, ncls),
        out_shape=jax.ShapeDtypeStruct((N_OBJ, ncls), jnp.float32),
        compiler_params=_TC_PARAMS,
    )(obj_h, parts, degp, wcls, b)


def kernel(roi_features, union_features, rel_pair_idxs, W_obj_down,
           W_rel_down, W_n2e, W_e2n, W_obj_cls, b_obj_cls, W_rel_cls,
           b_rel_cls):
    src = rel_pair_idxs[:, 0].astype(jnp.int32).reshape(NW * NCH, CH)
    dst = rel_pair_idxs[:, 1].astype(jnp.int32).reshape(NW * NCH, CH)
    z128 = jnp.zeros((CH, HID), jnp.float32)
    z16 = jnp.zeros((CH, DEGW), jnp.float32)
    ones16 = jnp.ones((CH, DEGW), jnp.float32)
    b_obj = b_obj_cls.reshape(1, -1)
    b_rel = b_rel_cls.reshape(1, -1)

    obj_h, p = _tc_prologue(roi_features, W_obj_down, W_n2e)

    gs, gd = _sc_gather(p, src, dst)
    rel_h, msg = _tc_edge0(union_features, W_rel_down, gs, gd, W_e2n)
    parts, degp = _sc_scatter(msg, src, dst, z128, z16, ones16)
    obj_h, p = _tc_node0(obj_h, parts, degp, W_n2e)

    gs, gd = _sc_gather(p, src, dst)
    msg, rel_logits = _tc_edge1(rel_h, gs, gd, W_e2n, W_rel_cls, b_rel)
    parts, degp = _sc_scatter(msg, src, dst, z128, z16, ones16)
    obj_logits = _tc_node1(obj_h, parts, degp, W_obj_cls, b_obj)

    return (obj_logits, rel_logits)
